# baseline (speedup 1.0000x reference)
import functools

import jax
import jax.numpy as jnp
from jax.experimental import pallas as pl
from jax.experimental.pallas import tpu as pltpu

N = 320000
D = 128
G = 256
B = 1280
NB = N // B


def _seg_mlp_kernel(ids_ref, x_ref, w1_ref, b1_ref, w2_ref, b2_ref,
                    out_ref, acc_ref, cnt_ref):
    i = pl.program_id(0)

    @pl.when(i == 0)
    def _init():
        acc_ref[...] = jnp.zeros_like(acc_ref)
        cnt_ref[...] = jnp.zeros_like(cnt_ref)

    ids = ids_ref[0, 0, :]
    x = x_ref[...]
    seg = jax.lax.broadcasted_iota(jnp.int32, (G, B), 0)
    onehot = (seg == ids[None, :]).astype(jnp.bfloat16)
    hi = x.astype(jnp.bfloat16)
    lo = (x - hi.astype(jnp.float32)).astype(jnp.bfloat16)
    acc_ref[...] += (
        jnp.dot(onehot, hi, preferred_element_type=jnp.float32)
        + jnp.dot(onehot, lo, preferred_element_type=jnp.float32))
    cnt_ref[...] += jnp.sum(onehot.astype(jnp.float32), axis=1)[None, :]

    @pl.when(i == NB - 1)
    def _finish():
        counts = jnp.maximum(cnt_ref[0, :], 1.0)
        means = acc_ref[...] / counts[:, None]
        h = means @ w1_ref[...] + b1_ref[0, :][None, :]
        h = 0.5 * h * (1.0 + jax.lax.erf(h * 0.7071067811865476))
        z = h @ w2_ref[...] + b2_ref[0, :][None, :]
        out_ref[...] = jax.nn.softplus(z)


@jax.jit
def kernel(bb_feats, segment_ids, W1, b1, W2, b2):
    ids3 = segment_ids.astype(jnp.int32).reshape(NB, 1, B)
    W2p = jnp.zeros((D, 128), W2.dtype).at[:, :6].set(W2)
    b2p = jnp.zeros((1, 128), b2.dtype).at[0, :6].set(b2)
    b1p = b1.reshape(1, D)

    out = pl.pallas_call(
        _seg_mlp_kernel,
        grid=(NB,),
        in_specs=[
            pl.BlockSpec((1, 1, B), lambda i: (i, 0, 0)),
            pl.BlockSpec((B, D), lambda i: (i, 0)),
            pl.BlockSpec((D, D), lambda i: (0, 0)),
            pl.BlockSpec((1, D), lambda i: (0, 0)),
            pl.BlockSpec((D, 128), lambda i: (0, 0)),
            pl.BlockSpec((1, 128), lambda i: (0, 0)),
        ],
        out_specs=pl.BlockSpec((G, 128), lambda i: (0, 0)),
        out_shape=jax.ShapeDtypeStruct((G, 128), jnp.float32),
        scratch_shapes=[
            pltpu.VMEM((G, D), jnp.float32),
            pltpu.VMEM((1, G), jnp.float32),
        ],
    )(ids3, bb_feats, W1, b1p, W2p, b2p)
    return out[:, :6]


# speedup vs baseline: 1.1908x; 1.1908x over previous
import jax
import jax.numpy as jnp
from jax.experimental import pallas as pl
from jax.experimental.pallas import tpu as pltpu

N = 320000
D = 128
G = 256
B = 1280
NB = N // B
W = 40


def _seg_mlp_kernel(ids_smem, ids_ref, x_ref, w1_ref, b1_ref, w2_ref, b2_ref,
                    out_ref, acc_ref, cnt_ref):
    i = pl.program_id(0)

    @pl.when(i == 0)
    def _init():
        acc_ref[...] = jnp.zeros_like(acc_ref)
        cnt_ref[...] = jnp.zeros_like(cnt_ref)

    ids = ids_ref[0, 0, :]
    x = x_ref[...]
    first = ids_smem[0, 0, 0]
    last = ids_smem[0, 0, B - 1]
    base = jnp.minimum((first // 8) * 8, G - W)

    @pl.when(last - base < W)
    def _local():
        seg = jax.lax.broadcasted_iota(jnp.int32, (W, B), 0)
        onehot = (seg == (ids - base)[None, :]).astype(jnp.float32)
        part = jnp.dot(onehot, x, preferred_element_type=jnp.float32)
        acc_ref[pl.ds(base, W), :] += part
        c = jnp.sum(onehot, axis=1)
        cnt_ref[pl.ds(base, W), :] += jnp.broadcast_to(c[:, None], (W, 128))

    @pl.when(last - base >= W)
    def _full():
        seg = jax.lax.broadcasted_iota(jnp.int32, (G, B), 0)
        onehot = (seg == ids[None, :]).astype(jnp.float32)
        acc_ref[...] += jnp.dot(onehot, x, preferred_element_type=jnp.float32)
        c = jnp.sum(onehot, axis=1)
        cnt_ref[...] += jnp.broadcast_to(c[:, None], (G, 128))

    @pl.when(i == NB - 1)
    def _finish():
        counts = jnp.maximum(cnt_ref[:, 0], 1.0)
        means = acc_ref[...] / counts[:, None]
        h = means @ w1_ref[...] + b1_ref[0, :][None, :]
        h = 0.5 * h * (1.0 + jax.lax.erf(h * 0.7071067811865476))
        z = h @ w2_ref[...] + b2_ref[0, :][None, :]
        out_ref[...] = jax.nn.softplus(z)


@jax.jit
def kernel(bb_feats, segment_ids, W1, b1, W2, b2):
    ids3 = segment_ids.astype(jnp.int32).reshape(NB, 1, B)
    W2p = jnp.zeros((D, 128), W2.dtype).at[:, :6].set(W2)
    b2p = jnp.zeros((1, 128), b2.dtype).at[0, :6].set(b2)
    b1p = b1.reshape(1, D)

    out = pl.pallas_call(
        _seg_mlp_kernel,
        grid=(NB,),
        in_specs=[
            pl.BlockSpec((1, 1, B), lambda i: (i, 0, 0),
                         memory_space=pltpu.SMEM),
            pl.BlockSpec((1, 1, B), lambda i: (i, 0, 0)),
            pl.BlockSpec((B, D), lambda i: (i, 0)),
            pl.BlockSpec((D, D), lambda i: (0, 0)),
            pl.BlockSpec((1, D), lambda i: (0, 0)),
            pl.BlockSpec((D, 128), lambda i: (0, 0)),
            pl.BlockSpec((1, 128), lambda i: (0, 0)),
        ],
        out_specs=pl.BlockSpec((G, 128), lambda i: (0, 0)),
        out_shape=jax.ShapeDtypeStruct((G, 128), jnp.float32),
        scratch_shapes=[
            pltpu.VMEM((G, D), jnp.float32),
            pltpu.VMEM((G, 128), jnp.float32),
        ],
    )(ids3, ids3, bb_feats, W1, b1p, W2p, b2p)
    return out[:, :6]


# speedup vs baseline: 1.8111x; 1.5209x over previous
import jax
import jax.numpy as jnp
from jax.experimental import pallas as pl
from jax.experimental.pallas import tpu as pltpu

N = 320000
D = 128
G = 256
B = 2560
NB = N // B
W = 48


def _seg_mlp_kernel(ids_smem, ids_ref, x_ref, w1_ref, b1_ref, w2_ref, b2_ref,
                    out_ref, acc_ref, cnt_ref):
    i = pl.program_id(0)

    @pl.when(i == 0)
    def _init():
        acc_ref[...] = jnp.zeros_like(acc_ref)
        cnt_ref[...] = jnp.zeros_like(cnt_ref)

    ids = ids_ref[0, 0, :]
    x = x_ref[...]
    first = ids_smem[0, 0, 0]
    last = ids_smem[0, 0, B - 1]
    base = jnp.minimum((first // 8) * 8, G - W)

    @pl.when(last - base < W)
    def _local():
        seg = jax.lax.broadcasted_iota(jnp.int32, (W, B), 0)
        onehot = (seg == (ids - base)[None, :]).astype(jnp.float32)
        part = jnp.dot(onehot, x, preferred_element_type=jnp.float32)
        acc_ref[pl.ds(base, W), :] += part
        c = jnp.sum(onehot, axis=1)
        cnt_ref[pl.ds(base, W), :] += jnp.broadcast_to(c[:, None], (W, 128))

    @pl.when(last - base >= W)
    def _full():
        seg = jax.lax.broadcasted_iota(jnp.int32, (G, B), 0)
        onehot = (seg == ids[None, :]).astype(jnp.float32)
        acc_ref[...] += jnp.dot(onehot, x, preferred_element_type=jnp.float32)
        c = jnp.sum(onehot, axis=1)
        cnt_ref[...] += jnp.broadcast_to(c[:, None], (G, 128))

    @pl.when(i == NB - 1)
    def _finish():
        counts = jnp.maximum(cnt_ref[:, 0], 1.0)
        means = acc_ref[...] / counts[:, None]
        h = means @ w1_ref[...] + b1_ref[0, :][None, :]
        h = 0.5 * h * (1.0 + jax.lax.erf(h * 0.7071067811865476))
        z = h @ w2_ref[...] + b2_ref[0, :][None, :]
        out_ref[...] = jax.nn.softplus(z)


@jax.jit
def kernel(bb_feats, segment_ids, W1, b1, W2, b2):
    ids3 = segment_ids.astype(jnp.int32).reshape(NB, 1, B)
    W2p = jnp.zeros((D, 128), W2.dtype).at[:, :6].set(W2)
    b2p = jnp.zeros((1, 128), b2.dtype).at[0, :6].set(b2)
    b1p = b1.reshape(1, D)

    out = pl.pallas_call(
        _seg_mlp_kernel,
        grid=(NB,),
        in_specs=[
            pl.BlockSpec((1, 1, B), lambda i: (i, 0, 0),
                         memory_space=pltpu.SMEM),
            pl.BlockSpec((1, 1, B), lambda i: (i, 0, 0)),
            pl.BlockSpec((B, D), lambda i: (i, 0)),
            pl.BlockSpec((D, D), lambda i: (0, 0)),
            pl.BlockSpec((1, D), lambda i: (0, 0)),
            pl.BlockSpec((D, 128), lambda i: (0, 0)),
            pl.BlockSpec((1, 128), lambda i: (0, 0)),
        ],
        out_specs=pl.BlockSpec((G, 128), lambda i: (0, 0)),
        out_shape=jax.ShapeDtypeStruct((G, 128), jnp.float32),
        scratch_shapes=[
            pltpu.VMEM((G, D), jnp.float32),
            pltpu.VMEM((G, 128), jnp.float32),
        ],
    )(ids3, ids3, bb_feats, W1, b1p, W2p, b2p)
    return out[:, :6]


# speedup vs baseline: 2.7799x; 1.5349x over previous
import jax
import jax.numpy as jnp
from jax.experimental import pallas as pl
from jax.experimental.pallas import tpu as pltpu

N = 320000
D = 128
G = 256
B = 6400
NB = N // B
W = 64


def _seg_mlp_kernel(ids_smem, ids_ref, x_ref, w1_ref, b1_ref, w2_ref, b2_ref,
                    out_ref, acc_ref, cnt_ref):
    i = pl.program_id(0)

    @pl.when(i == 0)
    def _init():
        acc_ref[...] = jnp.zeros_like(acc_ref)
        cnt_ref[...] = jnp.zeros_like(cnt_ref)

    ids = ids_ref[0, 0, :]
    x = x_ref[...]
    first = ids_smem[0, 0, 0]
    last = ids_smem[0, 0, B - 1]
    base = jnp.minimum((first // 8) * 8, G - W)

    @pl.when(last - base < W)
    def _local():
        seg = jax.lax.broadcasted_iota(jnp.int32, (W, B), 0)
        onehot = (seg == (ids - base)[None, :]).astype(jnp.float32)
        part = jnp.dot(onehot, x, preferred_element_type=jnp.float32)
        acc_ref[pl.ds(base, W), :] += part
        c = jnp.sum(onehot, axis=1)
        cnt_ref[pl.ds(base, W), :] += jnp.broadcast_to(c[:, None], (W, 128))

    @pl.when(last - base >= W)
    def _full():
        seg = jax.lax.broadcasted_iota(jnp.int32, (G, B), 0)
        onehot = (seg == ids[None, :]).astype(jnp.float32)
        acc_ref[...] += jnp.dot(onehot, x, preferred_element_type=jnp.float32)
        c = jnp.sum(onehot, axis=1)
        cnt_ref[...] += jnp.broadcast_to(c[:, None], (G, 128))

    @pl.when(i == NB - 1)
    def _finish():
        counts = jnp.maximum(cnt_ref[:, 0], 1.0)
        means = acc_ref[...] / counts[:, None]
        h = means @ w1_ref[...] + b1_ref[0, :][None, :]
        h = 0.5 * h * (1.0 + jax.lax.erf(h * 0.7071067811865476))
        z = h @ w2_ref[...] + b2_ref[0, :][None, :]
        out_ref[...] = jax.nn.softplus(z)


@jax.jit
def kernel(bb_feats, segment_ids, W1, b1, W2, b2):
    ids3 = segment_ids.astype(jnp.int32).reshape(NB, 1, B)
    W2p = jnp.zeros((D, 128), W2.dtype).at[:, :6].set(W2)
    b2p = jnp.zeros((1, 128), b2.dtype).at[0, :6].set(b2)
    b1p = b1.reshape(1, D)

    out = pl.pallas_call(
        _seg_mlp_kernel,
        grid=(NB,),
        in_specs=[
            pl.BlockSpec((1, 1, B), lambda i: (i, 0, 0),
                         memory_space=pltpu.SMEM),
            pl.BlockSpec((1, 1, B), lambda i: (i, 0, 0)),
            pl.BlockSpec((B, D), lambda i: (i, 0)),
            pl.BlockSpec((D, D), lambda i: (0, 0)),
            pl.BlockSpec((1, D), lambda i: (0, 0)),
            pl.BlockSpec((D, 128), lambda i: (0, 0)),
            pl.BlockSpec((1, 128), lambda i: (0, 0)),
        ],
        out_specs=pl.BlockSpec((G, 128), lambda i: (0, 0)),
        out_shape=jax.ShapeDtypeStruct((G, 128), jnp.float32),
        scratch_shapes=[
            pltpu.VMEM((G, D), jnp.float32),
            pltpu.VMEM((G, 128), jnp.float32),
        ],
    )(ids3, ids3, bb_feats, W1, b1p, W2p, b2p)
    return out[:, :6]


# speedup vs baseline: 3.3206x; 1.1945x over previous
import jax
import jax.numpy as jnp
from jax.experimental import pallas as pl
from jax.experimental.pallas import tpu as pltpu

N = 320000
D = 128
G = 256
B = 12800
NB = N // B
W = 80


def _seg_mlp_kernel(ids_smem, ids_ref, x_ref, w1_ref, b1_ref, w2_ref, b2_ref,
                    out_ref, acc_ref, cnt_ref):
    i = pl.program_id(0)

    @pl.when(i == 0)
    def _init():
        acc_ref[...] = jnp.zeros_like(acc_ref)
        cnt_ref[...] = jnp.zeros_like(cnt_ref)

    ids = ids_ref[0, 0, :]
    x = x_ref[...]
    first = ids_smem[0, 0, 0]
    last = ids_smem[0, 0, B - 1]
    base = jnp.minimum((first // 8) * 8, G - W)

    @pl.when(last - base < W)
    def _local():
        seg = jax.lax.broadcasted_iota(jnp.int32, (W, B), 0)
        onehot = (seg == (ids - base)[None, :]).astype(jnp.float32)
        part = jnp.dot(onehot, x, preferred_element_type=jnp.float32)
        acc_ref[pl.ds(base, W), :] += part
        c = jnp.sum(onehot, axis=1)
        cnt_ref[pl.ds(base, W), :] += jnp.broadcast_to(c[:, None], (W, 128))

    @pl.when(last - base >= W)
    def _full():
        seg = jax.lax.broadcasted_iota(jnp.int32, (G, B), 0)
        onehot = (seg == ids[None, :]).astype(jnp.float32)
        acc_ref[...] += jnp.dot(onehot, x, preferred_element_type=jnp.float32)
        c = jnp.sum(onehot, axis=1)
        cnt_ref[...] += jnp.broadcast_to(c[:, None], (G, 128))

    @pl.when(i == NB - 1)
    def _finish():
        counts = jnp.maximum(cnt_ref[:, 0], 1.0)
        means = acc_ref[...] / counts[:, None]
        h = means @ w1_ref[...] + b1_ref[0, :][None, :]
        h = 0.5 * h * (1.0 + jax.lax.erf(h * 0.7071067811865476))
        z = h @ w2_ref[...] + b2_ref[0, :][None, :]
        out_ref[...] = jax.nn.softplus(z)


@jax.jit
def kernel(bb_feats, segment_ids, W1, b1, W2, b2):
    ids3 = segment_ids.astype(jnp.int32).reshape(NB, 1, B)
    W2p = jnp.zeros((D, 128), W2.dtype).at[:, :6].set(W2)
    b2p = jnp.zeros((1, 128), b2.dtype).at[0, :6].set(b2)
    b1p = b1.reshape(1, D)

    out = pl.pallas_call(
        _seg_mlp_kernel,
        grid=(NB,),
        in_specs=[
            pl.BlockSpec((1, 1, B), lambda i: (i, 0, 0),
                         memory_space=pltpu.SMEM),
            pl.BlockSpec((1, 1, B), lambda i: (i, 0, 0)),
            pl.BlockSpec((B, D), lambda i: (i, 0)),
            pl.BlockSpec((D, D), lambda i: (0, 0)),
            pl.BlockSpec((1, D), lambda i: (0, 0)),
            pl.BlockSpec((D, 128), lambda i: (0, 0)),
            pl.BlockSpec((1, 128), lambda i: (0, 0)),
        ],
        out_specs=pl.BlockSpec((G, 128), lambda i: (0, 0)),
        out_shape=jax.ShapeDtypeStruct((G, 128), jnp.float32),
        scratch_shapes=[
            pltpu.VMEM((G, D), jnp.float32),
            pltpu.VMEM((G, 128), jnp.float32),
        ],
    )(ids3, ids3, bb_feats, W1, b1p, W2p, b2p)
    return out[:, :6]


# speedup vs baseline: 3.4206x; 1.0301x over previous
import jax
import jax.numpy as jnp
from jax.experimental import pallas as pl
from jax.experimental.pallas import tpu as pltpu

N = 320000
D = 128
G = 256
B = 16000
NB = N // B
W = 96


def _seg_mlp_kernel(ids_smem, ids_ref, x_ref, w1_ref, b1_ref, w2_ref, b2_ref,
                    out_ref, acc_ref, cnt_ref):
    i = pl.program_id(0)

    @pl.when(i == 0)
    def _init():
        acc_ref[...] = jnp.zeros_like(acc_ref)
        cnt_ref[...] = jnp.zeros_like(cnt_ref)

    ids = ids_ref[0, 0, :]
    x = x_ref[...]
    first = ids_smem[0, 0, 0]
    last = ids_smem[0, 0, B - 1]
    base = jnp.minimum((first // 8) * 8, G - W)

    @pl.when(last - base < W)
    def _local():
        seg = jax.lax.broadcasted_iota(jnp.int32, (W, B), 0)
        onehot = (seg == (ids - base)[None, :]).astype(jnp.float32)
        part = jnp.dot(onehot, x, preferred_element_type=jnp.float32)
        acc_ref[pl.ds(base, W), :] += part
        c = jnp.sum(onehot, axis=1)
        cnt_ref[pl.ds(base, W), :] += jnp.broadcast_to(c[:, None], (W, 128))

    @pl.when(last - base >= W)
    def _full():
        seg = jax.lax.broadcasted_iota(jnp.int32, (G, B), 0)
        onehot = (seg == ids[None, :]).astype(jnp.float32)
        acc_ref[...] += jnp.dot(onehot, x, preferred_element_type=jnp.float32)
        c = jnp.sum(onehot, axis=1)
        cnt_ref[...] += jnp.broadcast_to(c[:, None], (G, 128))

    @pl.when(i == NB - 1)
    def _finish():
        counts = jnp.maximum(cnt_ref[:, 0], 1.0)
        means = acc_ref[...] / counts[:, None]
        h = means @ w1_ref[...] + b1_ref[0, :][None, :]
        h = 0.5 * h * (1.0 + jax.lax.erf(h * 0.7071067811865476))
        z = h @ w2_ref[...] + b2_ref[0, :][None, :]
        out_ref[...] = jax.nn.softplus(z)


@jax.jit
def kernel(bb_feats, segment_ids, W1, b1, W2, b2):
    ids3 = segment_ids.astype(jnp.int32).reshape(NB, 1, B)
    W2p = jnp.zeros((D, 128), W2.dtype).at[:, :6].set(W2)
    b2p = jnp.zeros((1, 128), b2.dtype).at[0, :6].set(b2)
    b1p = b1.reshape(1, D)

    out = pl.pallas_call(
        _seg_mlp_kernel,
        grid=(NB,),
        in_specs=[
            pl.BlockSpec((1, 1, B), lambda i: (i, 0, 0),
                         memory_space=pltpu.SMEM),
            pl.BlockSpec((1, 1, B), lambda i: (i, 0, 0)),
            pl.BlockSpec((B, D), lambda i: (i, 0)),
            pl.BlockSpec((D, D), lambda i: (0, 0)),
            pl.BlockSpec((1, D), lambda i: (0, 0)),
            pl.BlockSpec((D, 128), lambda i: (0, 0)),
            pl.BlockSpec((1, 128), lambda i: (0, 0)),
        ],
        out_specs=pl.BlockSpec((G, 128), lambda i: (0, 0)),
        out_shape=jax.ShapeDtypeStruct((G, 128), jnp.float32),
        scratch_shapes=[
            pltpu.VMEM((G, D), jnp.float32),
            pltpu.VMEM((G, 128), jnp.float32),
        ],
    )(ids3, ids3, bb_feats, W1, b1p, W2p, b2p)
    return out[:, :6]


# speedup vs baseline: 3.6564x; 1.0689x over previous
import jax
import jax.numpy as jnp
from jax.experimental import pallas as pl
from jax.experimental.pallas import tpu as pltpu

N = 320000
D = 128
G = 256
B = 16000
NB = N // B
W = 32


def _seg_mlp_kernel(ids_smem, ids_ref, x_ref, w1_ref, b1_ref, w2_ref, b2_ref,
                    out_ref, acc_ref, cnt_ref):
    i = pl.program_id(0)

    @pl.when(i == 0)
    def _init():
        acc_ref[...] = jnp.zeros_like(acc_ref)
        cnt_ref[...] = jnp.zeros_like(cnt_ref)

    ids = ids_ref[0, 0, :]
    x = x_ref[...]
    first = ids_smem[0, 0, 0]
    last = ids_smem[0, 0, B - 1]
    base = jnp.minimum((first // 8) * 8, G - W)

    @pl.when(last - base < W)
    def _local():
        seg = jax.lax.broadcasted_iota(jnp.int32, (W, B), 0)
        onehot = (seg == (ids - base)[None, :]).astype(jnp.float32)
        part = jnp.dot(onehot, x, preferred_element_type=jnp.float32)
        acc_ref[pl.ds(base, W), :] += part
        c = jnp.sum(onehot, axis=1)
        cnt_ref[pl.ds(base, W), :] += jnp.broadcast_to(c[:, None], (W, 128))

    @pl.when(last - base >= W)
    def _full():
        seg = jax.lax.broadcasted_iota(jnp.int32, (G, B), 0)
        onehot = (seg == ids[None, :]).astype(jnp.float32)
        acc_ref[...] += jnp.dot(onehot, x, preferred_element_type=jnp.float32)
        c = jnp.sum(onehot, axis=1)
        cnt_ref[...] += jnp.broadcast_to(c[:, None], (G, 128))

    @pl.when(i == NB - 1)
    def _finish():
        counts = jnp.maximum(cnt_ref[:, 0], 1.0)
        means = acc_ref[...] / counts[:, None]
        h = means @ w1_ref[...] + b1_ref[0, :][None, :]
        h = 0.5 * h * (1.0 + jax.lax.erf(h * 0.7071067811865476))
        z = h @ w2_ref[...] + b2_ref[0, :][None, :]
        out_ref[...] = jax.nn.softplus(z)


@jax.jit
def kernel(bb_feats, segment_ids, W1, b1, W2, b2):
    ids3 = segment_ids.astype(jnp.int32).reshape(NB, 1, B)
    W2p = jnp.zeros((D, 128), W2.dtype).at[:, :6].set(W2)
    b2p = jnp.zeros((1, 128), b2.dtype).at[0, :6].set(b2)
    b1p = b1.reshape(1, D)

    out = pl.pallas_call(
        _seg_mlp_kernel,
        grid=(NB,),
        in_specs=[
            pl.BlockSpec((1, 1, B), lambda i: (i, 0, 0),
                         memory_space=pltpu.SMEM),
            pl.BlockSpec((1, 1, B), lambda i: (i, 0, 0)),
            pl.BlockSpec((B, D), lambda i: (i, 0)),
            pl.BlockSpec((D, D), lambda i: (0, 0)),
            pl.BlockSpec((1, D), lambda i: (0, 0)),
            pl.BlockSpec((D, 128), lambda i: (0, 0)),
            pl.BlockSpec((1, 128), lambda i: (0, 0)),
        ],
        out_specs=pl.BlockSpec((G, 128), lambda i: (0, 0)),
        out_shape=jax.ShapeDtypeStruct((G, 128), jnp.float32),
        scratch_shapes=[
            pltpu.VMEM((G, D), jnp.float32),
            pltpu.VMEM((G, 128), jnp.float32),
        ],
    )(ids3, ids3, bb_feats, W1, b1p, W2p, b2p)
    return out[:, :6]
